# Initial kernel scaffold; baseline (speedup 1.0000x reference)
import jax
import jax.numpy as jnp
from jax.experimental import pallas as pl


def kernel(t, t_section, log_lambda, breakpoints, widths):
    raise NotImplementedError("write your pallas kernel here")


# speedup vs baseline: 26.6420x; 26.6420x over previous
import functools

import jax
import jax.numpy as jnp
from jax import lax
from jax.experimental import pallas as pl
from jax.experimental.pallas import tpu as pltpu
from jax.experimental.pallas import tpu_sc as plsc

L = 16
NC = 2
NS = 16
NW = NC * NS
KP = 112


def _hazard_body(t_hbm, s_hbm, ll_hbm, bp_hbm, w_hbm, llo_hbm, ch_hbm,
                 t_v, s_v, ll_t, bp_t, w_t, lam_t, cum_t, o1_v, o2_v, sem,
                 *, chunk):
    wid = lax.axis_index("s") * NC + lax.axis_index("c")
    base = wid * chunk

    cp_t = pltpu.async_copy(t_hbm.at[pl.ds(base, chunk)], t_v, sem)
    cp_s = pltpu.async_copy(s_hbm.at[pl.ds(base, chunk)], s_v, sem)
    cp_ll = pltpu.async_copy(ll_hbm, ll_t, sem)
    cp_bp = pltpu.async_copy(bp_hbm, bp_t, sem)
    cp_w = pltpu.async_copy(w_hbm, w_t, sem)
    cp_t.wait()
    cp_s.wait()
    cp_ll.wait()
    cp_bp.wait()
    cp_w.wait()

    lane = lax.iota(jnp.int32, L)
    dn = lax.GatherDimensionNumbers(
        offset_dims=(), collapsed_slice_dims=(0,), start_index_map=(0,))

    def lane_gather(x, idx):
        return lax.gather(x, idx[:, None], dn, slice_sizes=(1,),
                          mode=lax.GatherScatterMode.PROMISE_IN_BOUNDS)

    def table_step(c, carry):
        sl = pl.ds(c * L, L)
        lam = jnp.exp(ll_t[sl])
        prod = lam * w_t[sl]
        incl = prod
        for d in (1, 2, 4, 8):
            shifted = lane_gather(incl, jnp.maximum(lane - d, 0))
            incl = incl + jnp.where(lane >= d, shifted, jnp.float32(0.0))
        lam_t[sl] = lam
        cum_t[sl] = (incl - prod) + carry
        total = lane_gather(incl, jnp.full((L,), L - 1, jnp.int32))
        return carry + total

    lax.fori_loop(0, KP // L, table_step, jnp.zeros((L,), jnp.float32),
                  unroll=True)

    def batch_step(i, carry):
        sl = pl.ds(i * L, L)
        s = s_v[sl]
        tt = t_v[sl]
        llg = plsc.load_gather(ll_t, [s])
        lamg = plsc.load_gather(lam_t, [s])
        cumg = plsc.load_gather(cum_t, [s])
        bpg = plsc.load_gather(bp_t, [s])
        o1_v[sl] = llg
        o2_v[sl] = cumg + lamg * (tt - bpg)
        return carry

    lax.fori_loop(0, chunk // L, batch_step, 0, unroll=4)

    pltpu.sync_copy(o1_v, llo_hbm.at[pl.ds(base, chunk)])
    pltpu.sync_copy(o2_v, ch_hbm.at[pl.ds(base, chunk)])


@jax.jit
def kernel(t, t_section, log_lambda, breakpoints, widths):
    b = t.shape[0]
    k = log_lambda.shape[0]
    chunk = b // NW

    pad = KP - k
    ll_p = jnp.pad(log_lambda[:, 0], (0, pad))
    bp_p = jnp.pad(breakpoints, (0, pad))
    w_p = jnp.pad(widths[:, 0], (0, pad))
    t_flat = t[:, 0]
    s = t_section.astype(jnp.int32)

    f32 = jnp.float32
    run = pl.kernel(
        functools.partial(_hazard_body, chunk=chunk),
        out_type=(
            jax.ShapeDtypeStruct((b,), f32),
            jax.ShapeDtypeStruct((b,), f32),
        ),
        mesh=plsc.VectorSubcoreMesh(
            core_axis_name="c", subcore_axis_name="s",
            num_cores=NC, num_subcores=NS,
        ),
        compiler_params=pltpu.CompilerParams(needs_layout_passes=False),
        scratch_types=[
            pltpu.VMEM((chunk,), f32),
            pltpu.VMEM((chunk,), jnp.int32),
            pltpu.VMEM((KP,), f32),
            pltpu.VMEM((KP,), f32),
            pltpu.VMEM((KP,), f32),
            pltpu.VMEM((KP,), f32),
            pltpu.VMEM((KP,), f32),
            pltpu.VMEM((chunk,), f32),
            pltpu.VMEM((chunk,), f32),
            pltpu.SemaphoreType.DMA,
        ],
    )
    llo, ch = run(t_flat, s, ll_p, bp_p, w_p)
    return llo[:, None], ch[:, None]
